# baseline (speedup 1.0000x reference)
import functools

import jax
import jax.numpy as jnp
from jax import lax
from jax.experimental import pallas as pl
from jax.experimental.pallas import tpu as pltpu
from jax.experimental.pallas import tpu_sc as plsc

DM = 1024
DI = 1024
NE = 64
KK = 2
NT = 2048
CAP = 256
BUF_ROWS = NE * CAP + CAP

NW = 32
TPW = NT // NW


def _gate_body(x_ref, wg_ref, bg_ref, i1_ref, i2_ref, sc1_ref, sc2_ref,
               d1_ref, d2_ref, s1_ref, s2_ref, cnt_ref):
    x = x_ref[...]
    logits = jax.lax.dot_general(
        x, wg_ref[...], (((1,), (0,)), ((), ())),
        preferred_element_type=jnp.float32) + bg_ref[...]
    ii = lax.broadcasted_iota(jnp.int32, (NT, NE), 1)
    v1 = jnp.max(logits, axis=1, keepdims=True)
    e1 = jnp.min(jnp.where(logits == v1, ii, NE + 1), axis=1, keepdims=True)
    l2 = jnp.where(ii == e1, -jnp.inf, logits)
    v2 = jnp.max(l2, axis=1, keepdims=True)
    e2 = jnp.min(jnp.where(l2 == v2, ii, NE + 1), axis=1, keepdims=True)
    t = jnp.exp(v2 - v1)
    s1 = 1.0 / (1.0 + t)
    s2 = 1.0 - s1
    oh1 = (ii == e1).astype(jnp.float32)
    oh2 = (ii == e2).astype(jnp.float32)
    m = oh1 + oh2
    NB, BR = NT // 128, 128
    mb = m.reshape(NB, BR, NE)
    s_blk = jnp.sum(mb, axis=1)
    bi_r = lax.broadcasted_iota(jnp.int32, (NB, NB), 0)
    bi_c = lax.broadcasted_iota(jnp.int32, (NB, NB), 1)
    ltri_b = (bi_c < bi_r).astype(jnp.float32)
    s_ex = jax.lax.dot_general(
        ltri_b, s_blk, (((1,), (0,)), ((), ())),
        preferred_element_type=jnp.float32)
    r_i = lax.broadcasted_iota(jnp.int32, (BR, BR), 0)
    c_i = lax.broadcasted_iota(jnp.int32, (BR, BR), 1)
    ltri = (c_i < r_i).astype(jnp.float32)
    pex_blocks = []
    for j in range(NB):
        pj = jax.lax.dot_general(
            ltri, mb[j], (((1,), (0,)), ((), ())),
            preferred_element_type=jnp.float32) + s_ex[j:j + 1, :]
        pex_blocks.append(pj)
    pex = jnp.concatenate(pex_blocks, axis=0)
    r1 = jnp.sum(pex * oh1, axis=1, keepdims=True).astype(jnp.int32)
    r2 = jnp.sum(pex * oh2, axis=1, keepdims=True).astype(jnp.int32)
    cnt = jnp.sum(m, axis=0, keepdims=True).astype(jnp.int32)
    ok1 = r1 < CAP
    ok2 = r2 < CAP
    row1 = e1 * CAP + r1
    row2 = e2 * CAP + r2
    i1_ref[...] = e1
    i2_ref[...] = e2
    sc1_ref[...] = jnp.where(ok1, s1, 0.0)
    sc2_ref[...] = jnp.where(ok2, s2, 0.0)
    d1_ref[...] = jnp.where(ok1, row1, NE * CAP)
    d2_ref[...] = jnp.where(ok2, row2, NE * CAP)
    s1_ref[...] = e1 * CAP + jnp.minimum(r1, CAP - 1)
    s2_ref[...] = e2 * CAP + jnp.minimum(r2, CAP - 1)
    cnt_ref[...] = cnt


def _gate(x, wg, bg):
    col_i = jax.ShapeDtypeStruct((NT, 1), jnp.int32)
    col_f = jax.ShapeDtypeStruct((NT, 1), jnp.float32)
    return pl.pallas_call(
        _gate_body,
        out_shape=(col_i, col_i, col_f, col_f, col_i, col_i, col_i, col_i,
                   jax.ShapeDtypeStruct((1, NE), jnp.int32)),
    )(x, wg, bg.reshape(1, NE))


def _dispatch_body(x_hbm, d1_hbm, d2_hbm, buf_hbm, idx_v, rows_v, sem):
    wid = lax.axis_index("s") * 2 + lax.axis_index("c")
    base = wid * TPW
    pltpu.sync_copy(x_hbm.at[pl.ds(base, TPW)], rows_v)
    pltpu.sync_copy(d1_hbm.at[pl.ds(base, TPW)], idx_v)
    pltpu.async_copy(rows_v, buf_hbm.at[idx_v], sem).wait()
    pltpu.sync_copy(d2_hbm.at[pl.ds(base, TPW)], idx_v)
    pltpu.async_copy(rows_v, buf_hbm.at[idx_v], sem).wait()


def _dispatch(x, d1, d2):
    mesh = plsc.VectorSubcoreMesh(core_axis_name="c", subcore_axis_name="s")
    return pl.kernel(
        _dispatch_body,
        mesh=mesh,
        out_type=jax.ShapeDtypeStruct((BUF_ROWS, DM), jnp.float32),
        scratch_types=[
            pltpu.VMEM((TPW,), jnp.int32),
            pltpu.VMEM((TPW, DM), jnp.float32),
            pltpu.SemaphoreType.DMA,
        ],
    )(x, d1, d2)


RT = 64
NST = CAP // RT


def _ffn_body(cnt_ref, buf_hbm, w1_ref, b1_ref, w2_ref, b2_ref, y_hbm,
              xb_v, y_v, lsem, ssem):
    e = pl.program_id(0)
    cnt = cnt_ref[0, e]

    def ld(j):
        return pltpu.make_async_copy(
            buf_hbm.at[pl.ds(e * CAP + j * RT, RT), :],
            xb_v.at[pl.ds(j * RT, RT), :], lsem.at[j])

    def st(j):
        return pltpu.make_async_copy(
            y_v.at[pl.ds(j * RT, RT), :],
            y_hbm.at[pl.ds(e * CAP + j * RT, RT), :], ssem.at[j])

    for j in range(NST):
        @pl.when(cnt > j * RT)
        def _(j=j):
            ld(j).start()

    for j in range(NST):
        @pl.when(cnt > j * RT)
        def _(j=j):
            ld(j).wait()
            h = jax.lax.dot_general(
                xb_v[pl.ds(j * RT, RT), :], w1_ref[0], (((1,), (0,)), ((), ())),
                preferred_element_type=jnp.float32) + b1_ref[0]
            h = jnp.maximum(h, 0.0)
            y = jax.lax.dot_general(
                h, w2_ref[0], (((1,), (0,)), ((), ())),
                preferred_element_type=jnp.float32) + b2_ref[0]
            y_v[pl.ds(j * RT, RT), :] = y
            st(j).start()

    for j in range(NST):
        @pl.when(cnt > j * RT)
        def _(j=j):
            st(j).wait()


def _ffn(cnt, buf, w1, b1, w2, b2):
    return pl.pallas_call(
        _ffn_body,
        grid=(NE,),
        in_specs=[
            pl.BlockSpec(memory_space=pltpu.SMEM),
            pl.BlockSpec(memory_space=pl.ANY),
            pl.BlockSpec((1, DM, DI), lambda e: (e, 0, 0)),
            pl.BlockSpec((1, 1, DI), lambda e: (e, 0, 0)),
            pl.BlockSpec((1, DI, DM), lambda e: (e, 0, 0)),
            pl.BlockSpec((1, 1, DM), lambda e: (e, 0, 0)),
        ],
        out_specs=pl.BlockSpec(memory_space=pl.ANY),
        out_shape=jax.ShapeDtypeStruct((NE * CAP, DM), jnp.float32),
        scratch_shapes=[
            pltpu.VMEM((CAP, DM), jnp.float32),
            pltpu.VMEM((CAP, DM), jnp.float32),
            pltpu.SemaphoreType.DMA((NST,)),
            pltpu.SemaphoreType.DMA((NST,)),
        ],
    )(cnt, buf, w1, b1.reshape(NE, 1, DI), w2, b2.reshape(NE, 1, DM))


def _cgather_body(y_hbm, s1_hbm, s2_hbm, g1_hbm, g2_hbm, idx_v, rows_v, sem):
    wid = lax.axis_index("s") * 2 + lax.axis_index("c")
    base = wid * TPW
    pltpu.sync_copy(s1_hbm.at[pl.ds(base, TPW)], idx_v)
    pltpu.async_copy(y_hbm.at[idx_v], rows_v, sem).wait()
    pltpu.sync_copy(rows_v, g1_hbm.at[pl.ds(base, TPW)])
    pltpu.sync_copy(s2_hbm.at[pl.ds(base, TPW)], idx_v)
    pltpu.async_copy(y_hbm.at[idx_v], rows_v, sem).wait()
    pltpu.sync_copy(rows_v, g2_hbm.at[pl.ds(base, TPW)])


def _cgather(y, s1, s2):
    mesh = plsc.VectorSubcoreMesh(core_axis_name="c", subcore_axis_name="s")
    out = jax.ShapeDtypeStruct((NT, DM), jnp.float32)
    return pl.kernel(
        _cgather_body,
        mesh=mesh,
        out_type=(out, out),
        scratch_types=[
            pltpu.VMEM((TPW,), jnp.int32),
            pltpu.VMEM((TPW, DM), jnp.float32),
            pltpu.SemaphoreType.DMA,
        ],
    )(y, s1, s2)


def _combine_body(x_ref, g1_ref, g2_ref, sc1_ref, sc2_ref, o_ref):
    o_ref[...] = (x_ref[...] + sc1_ref[...] * g1_ref[...]
                  + sc2_ref[...] * g2_ref[...])


def _combine(x, g1, g2, sc1, sc2):
    return pl.pallas_call(
        _combine_body,
        out_shape=jax.ShapeDtypeStruct((NT, DM), jnp.float32),
    )(x, g1, g2, sc1, sc2)


def kernel(x, Wg, bg, W1, b1, W2, b2):
    (_, _, sc1, sc2, d1, d2, s1, s2, cnt) = _gate(x, Wg, bg)
    buf = _dispatch(x, d1.reshape(NT), d2.reshape(NT))
    y = _ffn(cnt, buf, W1, b1, W2, b2)
    g1, g2 = _cgather(y, s1.reshape(NT), s2.reshape(NT))
    return _combine(x, g1, g2, sc1, sc2)


# speedup vs baseline: 1.7136x; 1.7136x over previous
import functools

import jax
import jax.numpy as jnp
from jax import lax
from jax.experimental import pallas as pl
from jax.experimental.pallas import tpu as pltpu
from jax.experimental.pallas import tpu_sc as plsc

DM = 1024
DI = 1024
NE = 64
KK = 2
NT = 2048
CAP = 256
BUF_ROWS = NE * CAP + CAP

NW = 32
TPW = NT // NW


def _gate_body(x_ref, wg_ref, bg_ref, i1_ref, i2_ref, sc1_ref, sc2_ref,
               d1_ref, d2_ref, s1_ref, s2_ref, cnt_ref):
    x = x_ref[...]
    logits = jax.lax.dot_general(
        x, wg_ref[...], (((1,), (0,)), ((), ())),
        preferred_element_type=jnp.float32) + bg_ref[...]
    ii = lax.broadcasted_iota(jnp.int32, (NT, NE), 1)
    v1 = jnp.max(logits, axis=1, keepdims=True)
    e1 = jnp.min(jnp.where(logits == v1, ii, NE + 1), axis=1, keepdims=True)
    l2 = jnp.where(ii == e1, -jnp.inf, logits)
    v2 = jnp.max(l2, axis=1, keepdims=True)
    e2 = jnp.min(jnp.where(l2 == v2, ii, NE + 1), axis=1, keepdims=True)
    t = jnp.exp(v2 - v1)
    s1 = 1.0 / (1.0 + t)
    s2 = 1.0 - s1
    oh1 = (ii == e1).astype(jnp.float32)
    oh2 = (ii == e2).astype(jnp.float32)
    m = oh1 + oh2
    NB, BR = NT // 128, 128
    mb = m.reshape(NB, BR, NE)
    s_blk = jnp.sum(mb, axis=1)
    bi_r = lax.broadcasted_iota(jnp.int32, (NB, NB), 0)
    bi_c = lax.broadcasted_iota(jnp.int32, (NB, NB), 1)
    ltri_b = (bi_c < bi_r).astype(jnp.float32)
    s_ex = jax.lax.dot_general(
        ltri_b, s_blk, (((1,), (0,)), ((), ())),
        preferred_element_type=jnp.float32)
    r_i = lax.broadcasted_iota(jnp.int32, (BR, BR), 0)
    c_i = lax.broadcasted_iota(jnp.int32, (BR, BR), 1)
    ltri = (c_i < r_i).astype(jnp.float32)
    pex_blocks = []
    for j in range(NB):
        pj = jax.lax.dot_general(
            ltri, mb[j], (((1,), (0,)), ((), ())),
            preferred_element_type=jnp.float32) + s_ex[j:j + 1, :]
        pex_blocks.append(pj)
    pex = jnp.concatenate(pex_blocks, axis=0)
    r1 = jnp.sum(pex * oh1, axis=1, keepdims=True).astype(jnp.int32)
    r2 = jnp.sum(pex * oh2, axis=1, keepdims=True).astype(jnp.int32)
    cnt = jnp.sum(m, axis=0, keepdims=True).astype(jnp.int32)
    ok1 = r1 < CAP
    ok2 = r2 < CAP
    row1 = e1 * CAP + r1
    row2 = e2 * CAP + r2
    i1_ref[...] = e1
    i2_ref[...] = e2
    sc1_ref[...] = jnp.where(ok1, s1, 0.0)
    sc2_ref[...] = jnp.where(ok2, s2, 0.0)
    d1_ref[...] = jnp.where(ok1, row1, NE * CAP)
    d2_ref[...] = jnp.where(ok2, row2, NE * CAP)
    s1_ref[...] = e1 * CAP + jnp.minimum(r1, CAP - 1)
    s2_ref[...] = e2 * CAP + jnp.minimum(r2, CAP - 1)
    cnt_ref[...] = cnt


def _gate(x, wg, bg):
    col_i = jax.ShapeDtypeStruct((NT, 1), jnp.int32)
    col_f = jax.ShapeDtypeStruct((NT, 1), jnp.float32)
    return pl.pallas_call(
        _gate_body,
        out_shape=(col_i, col_i, col_f, col_f, col_i, col_i, col_i, col_i,
                   jax.ShapeDtypeStruct((1, NE), jnp.int32)),
    )(x, wg, bg.reshape(1, NE))


def _dispatch_body(x_hbm, d1_hbm, d2_hbm, buf_hbm, idx_v, rows_v, sem):
    wid = lax.axis_index("s") * 2 + lax.axis_index("c")
    base = wid * TPW
    pltpu.sync_copy(x_hbm.at[pl.ds(base, TPW)], rows_v)
    pltpu.sync_copy(d1_hbm.at[pl.ds(base, TPW)], idx_v)
    pltpu.async_copy(rows_v, buf_hbm.at[idx_v], sem).wait()
    pltpu.sync_copy(d2_hbm.at[pl.ds(base, TPW)], idx_v)
    pltpu.async_copy(rows_v, buf_hbm.at[idx_v], sem).wait()


def _dispatch(x, d1, d2):
    mesh = plsc.VectorSubcoreMesh(core_axis_name="c", subcore_axis_name="s")
    return pl.kernel(
        _dispatch_body,
        mesh=mesh,
        out_type=jax.ShapeDtypeStruct((BUF_ROWS, DM), jnp.float32),
        scratch_types=[
            pltpu.VMEM((TPW,), jnp.int32),
            pltpu.VMEM((TPW, DM), jnp.float32),
            pltpu.SemaphoreType.DMA,
        ],
    )(x, d1, d2)


RT = 64
NST = CAP // RT


def _ffn_body(cnt_ref, buf_hbm, w1_ref, b1_ref, w2_ref, b2_ref, y_hbm,
              xb_v, y_v, lsem, ssem):
    e = pl.program_id(0)
    slot = lax.rem(e, 2)

    def ld(ee, s, j):
        return pltpu.make_async_copy(
            buf_hbm.at[pl.ds(ee * CAP + j * RT, RT), :],
            xb_v.at[s, pl.ds(j * RT, RT), :], lsem.at[s, j])

    def st(ee, s, j):
        return pltpu.make_async_copy(
            y_v.at[s, pl.ds(j * RT, RT), :],
            y_hbm.at[pl.ds(ee * CAP + j * RT, RT), :], ssem.at[s, j])

    def issue_loads(ee, s):
        cn = cnt_ref[0, ee]
        for j in range(NST):
            @pl.when(cn > j * RT)
            def _(j=j):
                ld(ee, s, j).start()

    @pl.when(e == 0)
    def _():
        issue_loads(0, 0)

    @pl.when(e + 1 < NE)
    def _():
        issue_loads(e + 1, 1 - slot)

    @pl.when(e >= 2)
    def _():
        cp = cnt_ref[0, e - 2]
        for j in range(NST):
            @pl.when(cp > j * RT)
            def _(j=j):
                st(e - 2, slot, j).wait()

    cnt = cnt_ref[0, e]
    for j in range(NST):
        @pl.when(cnt > j * RT)
        def _(j=j):
            ld(e, slot, j).wait()
            h = jax.lax.dot_general(
                xb_v[slot, pl.ds(j * RT, RT), :], w1_ref[0],
                (((1,), (0,)), ((), ())),
                preferred_element_type=jnp.float32) + b1_ref[0]
            h = jnp.maximum(h, 0.0)
            y = jax.lax.dot_general(
                h, w2_ref[0], (((1,), (0,)), ((), ())),
                preferred_element_type=jnp.float32) + b2_ref[0]
            y_v[slot, pl.ds(j * RT, RT), :] = y
            st(e, slot, j).start()

    @pl.when(e == NE - 1)
    def _():
        for ee in (NE - 2, NE - 1):
            cp = cnt_ref[0, ee]
            for j in range(NST):
                @pl.when(cp > j * RT)
                def _(ee=ee, j=j):
                    st(ee, ee % 2, j).wait()


def _ffn(cnt, buf, w1, b1, w2, b2):
    return pl.pallas_call(
        _ffn_body,
        grid=(NE,),
        in_specs=[
            pl.BlockSpec(memory_space=pltpu.SMEM),
            pl.BlockSpec(memory_space=pl.ANY),
            pl.BlockSpec((1, DM, DI), lambda e: (e, 0, 0)),
            pl.BlockSpec((1, 1, DI), lambda e: (e, 0, 0)),
            pl.BlockSpec((1, DI, DM), lambda e: (e, 0, 0)),
            pl.BlockSpec((1, 1, DM), lambda e: (e, 0, 0)),
        ],
        out_specs=pl.BlockSpec(memory_space=pl.ANY),
        out_shape=jax.ShapeDtypeStruct((NE * CAP, DM), jnp.float32),
        scratch_shapes=[
            pltpu.VMEM((2, CAP, DM), jnp.float32),
            pltpu.VMEM((2, CAP, DM), jnp.float32),
            pltpu.SemaphoreType.DMA((2, NST)),
            pltpu.SemaphoreType.DMA((2, NST)),
        ],
    )(cnt, buf, w1, b1.reshape(NE, 1, DI), w2, b2.reshape(NE, 1, DM))


def _cgather_body(y_hbm, s1_hbm, s2_hbm, g1_hbm, g2_hbm, idx_v, rows_v, sem):
    wid = lax.axis_index("s") * 2 + lax.axis_index("c")
    base = wid * TPW
    pltpu.sync_copy(s1_hbm.at[pl.ds(base, TPW)], idx_v)
    pltpu.async_copy(y_hbm.at[idx_v], rows_v, sem).wait()
    pltpu.sync_copy(rows_v, g1_hbm.at[pl.ds(base, TPW)])
    pltpu.sync_copy(s2_hbm.at[pl.ds(base, TPW)], idx_v)
    pltpu.async_copy(y_hbm.at[idx_v], rows_v, sem).wait()
    pltpu.sync_copy(rows_v, g2_hbm.at[pl.ds(base, TPW)])


def _cgather(y, s1, s2):
    mesh = plsc.VectorSubcoreMesh(core_axis_name="c", subcore_axis_name="s")
    out = jax.ShapeDtypeStruct((NT, DM), jnp.float32)
    return pl.kernel(
        _cgather_body,
        mesh=mesh,
        out_type=(out, out),
        scratch_types=[
            pltpu.VMEM((TPW,), jnp.int32),
            pltpu.VMEM((TPW, DM), jnp.float32),
            pltpu.SemaphoreType.DMA,
        ],
    )(y, s1, s2)


def _combine_body(x_ref, g1_ref, g2_ref, sc1_ref, sc2_ref, o_ref):
    o_ref[...] = (x_ref[...] + sc1_ref[...] * g1_ref[...]
                  + sc2_ref[...] * g2_ref[...])


def _combine(x, g1, g2, sc1, sc2):
    return pl.pallas_call(
        _combine_body,
        out_shape=jax.ShapeDtypeStruct((NT, DM), jnp.float32),
    )(x, g1, g2, sc1, sc2)


def kernel(x, Wg, bg, W1, b1, W2, b2):
    (_, _, sc1, sc2, d1, d2, s1, s2, cnt) = _gate(x, Wg, bg)
    buf = _dispatch(x, d1.reshape(NT), d2.reshape(NT))
    y = _ffn(cnt, buf, W1, b1, W2, b2)
    g1, g2 = _cgather(y, s1.reshape(NT), s2.reshape(NT))
    return _combine(x, g1, g2, sc1, sc2)


# speedup vs baseline: 1.7196x; 1.0035x over previous
import functools

import jax
import jax.numpy as jnp
from jax import lax
from jax.experimental import pallas as pl
from jax.experimental.pallas import tpu as pltpu
from jax.experimental.pallas import tpu_sc as plsc

DM = 1024
DI = 1024
NE = 64
KK = 2
NT = 2048
CAP = 256
BUF_ROWS = NE * CAP + CAP

NW = 32
TPW = NT // NW


GB = 512


def _gate_body(x_ref, wg_ref, bg_ref, sc1_ref, sc2_ref,
               d1_ref, d2_ref, s1_ref, s2_ref, cnt_ref, carry_ref):
    g = pl.program_id(0)

    @pl.when(g == 0)
    def _():
        carry_ref[...] = jnp.zeros((1, NE), jnp.float32)

    x = x_ref[...]
    logits = jax.lax.dot_general(
        x, wg_ref[...], (((1,), (0,)), ((), ())),
        preferred_element_type=jnp.float32) + bg_ref[...]
    ii = lax.broadcasted_iota(jnp.int32, (GB, NE), 1)
    v1 = jnp.max(logits, axis=1, keepdims=True)
    e1 = jnp.min(jnp.where(logits == v1, ii, NE + 1), axis=1, keepdims=True)
    l2 = jnp.where(ii == e1, -jnp.inf, logits)
    v2 = jnp.max(l2, axis=1, keepdims=True)
    e2 = jnp.min(jnp.where(l2 == v2, ii, NE + 1), axis=1, keepdims=True)
    t = jnp.exp(v2 - v1)
    s1 = 1.0 / (1.0 + t)
    s2 = 1.0 - s1
    oh1 = (ii == e1).astype(jnp.float32)
    oh2 = (ii == e2).astype(jnp.float32)
    m = oh1 + oh2
    NB, BR = GB // 128, 128
    mb = m.reshape(NB, BR, NE)
    s_blk = jnp.sum(mb, axis=1)
    bi_r = lax.broadcasted_iota(jnp.int32, (NB, NB), 0)
    bi_c = lax.broadcasted_iota(jnp.int32, (NB, NB), 1)
    ltri_b = (bi_c < bi_r).astype(jnp.float32)
    s_ex = jax.lax.dot_general(
        ltri_b, s_blk, (((1,), (0,)), ((), ())),
        preferred_element_type=jnp.float32)
    r_i = lax.broadcasted_iota(jnp.int32, (BR, BR), 0)
    c_i = lax.broadcasted_iota(jnp.int32, (BR, BR), 1)
    ltri = (c_i < r_i).astype(jnp.float32)
    carry = carry_ref[...]
    pex_blocks = []
    for j in range(NB):
        pj = jax.lax.dot_general(
            ltri, mb[j], (((1,), (0,)), ((), ())),
            preferred_element_type=jnp.float32) + (s_ex[j:j + 1, :] + carry)
        pex_blocks.append(pj)
    pex = jnp.concatenate(pex_blocks, axis=0)
    new_carry = carry + s_ex[NB - 1:NB, :] + s_blk[NB - 1:NB, :]
    carry_ref[...] = new_carry
    r1 = jnp.sum(pex * oh1, axis=1, keepdims=True).astype(jnp.int32)
    r2 = jnp.sum(pex * oh2, axis=1, keepdims=True).astype(jnp.int32)
    ok1 = r1 < CAP
    ok2 = r2 < CAP
    row1 = e1 * CAP + r1
    row2 = e2 * CAP + r2
    sc1_ref[...] = jnp.where(ok1, s1, 0.0)
    sc2_ref[...] = jnp.where(ok2, s2, 0.0)
    d1_ref[...] = jnp.where(ok1, row1, NE * CAP)
    d2_ref[...] = jnp.where(ok2, row2, NE * CAP)
    s1_ref[...] = e1 * CAP + jnp.minimum(r1, CAP - 1)
    s2_ref[...] = e2 * CAP + jnp.minimum(r2, CAP - 1)
    cnt_ref[...] = new_carry.astype(jnp.int32)


def _gate(x, wg, bg):
    ng = NT // GB
    col_i = jax.ShapeDtypeStruct((NT, 1), jnp.int32)
    col_f = jax.ShapeDtypeStruct((NT, 1), jnp.float32)
    col_spec_i = pl.BlockSpec((GB, 1), lambda g: (g, 0))
    return pl.pallas_call(
        _gate_body,
        grid=(ng,),
        in_specs=[
            pl.BlockSpec((GB, DM), lambda g: (g, 0)),
            pl.BlockSpec((DM, NE), lambda g: (0, 0)),
            pl.BlockSpec((1, NE), lambda g: (0, 0)),
        ],
        out_specs=(col_spec_i, col_spec_i, col_spec_i, col_spec_i,
                   col_spec_i, col_spec_i, pl.BlockSpec((1, NE), lambda g: (0, 0))),
        out_shape=(col_f, col_f, col_i, col_i, col_i, col_i,
                   jax.ShapeDtypeStruct((1, NE), jnp.int32)),
        scratch_shapes=[pltpu.VMEM((1, NE), jnp.float32)],
    )(x, wg, bg.reshape(1, NE))


def _dispatch_body(x_hbm, d1_hbm, d2_hbm, buf_hbm, idx_v, rows_v, sem):
    wid = lax.axis_index("s") * 2 + lax.axis_index("c")
    base = wid * TPW
    pltpu.sync_copy(x_hbm.at[pl.ds(base, TPW)], rows_v)
    pltpu.sync_copy(d1_hbm.at[pl.ds(base, TPW)], idx_v)
    pltpu.async_copy(rows_v, buf_hbm.at[idx_v], sem).wait()
    pltpu.sync_copy(d2_hbm.at[pl.ds(base, TPW)], idx_v)
    pltpu.async_copy(rows_v, buf_hbm.at[idx_v], sem).wait()


def _dispatch(x, d1, d2):
    mesh = plsc.VectorSubcoreMesh(core_axis_name="c", subcore_axis_name="s")
    return pl.kernel(
        _dispatch_body,
        mesh=mesh,
        out_type=jax.ShapeDtypeStruct((BUF_ROWS, DM), jnp.float32),
        scratch_types=[
            pltpu.VMEM((TPW,), jnp.int32),
            pltpu.VMEM((TPW, DM), jnp.float32),
            pltpu.SemaphoreType.DMA,
        ],
    )(x, d1, d2)


RT = 64
NST = CAP // RT


def _ffn_body(cnt_ref, buf_hbm, w1_ref, b1_ref, w2_ref, b2_ref, y_hbm,
              xb_v, y_v, lsem, ssem):
    e = pl.program_id(0)
    slot = lax.rem(e, 2)

    def ld(ee, s, j):
        return pltpu.make_async_copy(
            buf_hbm.at[pl.ds(ee * CAP + j * RT, RT), :],
            xb_v.at[s, pl.ds(j * RT, RT), :], lsem.at[s, j])

    def st(ee, s, j):
        return pltpu.make_async_copy(
            y_v.at[s, pl.ds(j * RT, RT), :],
            y_hbm.at[pl.ds(ee * CAP + j * RT, RT), :], ssem.at[s, j])

    def issue_loads(ee, s):
        cn = cnt_ref[0, ee]
        for j in range(NST):
            @pl.when(cn > j * RT)
            def _(j=j):
                ld(ee, s, j).start()

    @pl.when(e == 0)
    def _():
        issue_loads(0, 0)

    @pl.when(e + 1 < NE)
    def _():
        issue_loads(e + 1, 1 - slot)

    @pl.when(e >= 2)
    def _():
        cp = cnt_ref[0, e - 2]
        for j in range(NST):
            @pl.when(cp > j * RT)
            def _(j=j):
                st(e - 2, slot, j).wait()

    cnt = cnt_ref[0, e]
    for j in range(NST):
        @pl.when(cnt > j * RT)
        def _(j=j):
            ld(e, slot, j).wait()
            h = jax.lax.dot_general(
                xb_v[slot, pl.ds(j * RT, RT), :], w1_ref[0],
                (((1,), (0,)), ((), ())),
                preferred_element_type=jnp.float32) + b1_ref[pl.ds(e, 1), :]
            h = jnp.maximum(h, 0.0)
            y = jax.lax.dot_general(
                h, w2_ref[0], (((1,), (0,)), ((), ())),
                preferred_element_type=jnp.float32) + b2_ref[pl.ds(e, 1), :]
            y_v[slot, pl.ds(j * RT, RT), :] = y
            st(e, slot, j).start()

    @pl.when(e == NE - 1)
    def _():
        for ee in (NE - 2, NE - 1):
            cp = cnt_ref[0, ee]
            for j in range(NST):
                @pl.when(cp > j * RT)
                def _(ee=ee, j=j):
                    st(ee, ee % 2, j).wait()


def _ffn(cnt, buf, w1, b1, w2, b2):
    return pl.pallas_call(
        _ffn_body,
        grid=(NE,),
        in_specs=[
            pl.BlockSpec(memory_space=pltpu.SMEM),
            pl.BlockSpec(memory_space=pl.ANY),
            pl.BlockSpec((1, DM, DI), lambda e: (e, 0, 0)),
            pl.BlockSpec((NE, DI), lambda e: (0, 0)),
            pl.BlockSpec((1, DI, DM), lambda e: (e, 0, 0)),
            pl.BlockSpec((NE, DM), lambda e: (0, 0)),
        ],
        out_specs=pl.BlockSpec(memory_space=pl.ANY),
        out_shape=jax.ShapeDtypeStruct((NE * CAP, DM), jnp.float32),
        scratch_shapes=[
            pltpu.VMEM((2, CAP, DM), jnp.float32),
            pltpu.VMEM((2, CAP, DM), jnp.float32),
            pltpu.SemaphoreType.DMA((2, NST)),
            pltpu.SemaphoreType.DMA((2, NST)),
        ],
    )(cnt, buf, w1, b1, w2, b2)


def _cgather_body(y_hbm, s1_hbm, s2_hbm, g1_hbm, g2_hbm, idx_v, rows_v, sem):
    wid = lax.axis_index("s") * 2 + lax.axis_index("c")
    base = wid * TPW
    pltpu.sync_copy(s1_hbm.at[pl.ds(base, TPW)], idx_v)
    pltpu.async_copy(y_hbm.at[idx_v], rows_v, sem).wait()
    pltpu.sync_copy(rows_v, g1_hbm.at[pl.ds(base, TPW)])
    pltpu.sync_copy(s2_hbm.at[pl.ds(base, TPW)], idx_v)
    pltpu.async_copy(y_hbm.at[idx_v], rows_v, sem).wait()
    pltpu.sync_copy(rows_v, g2_hbm.at[pl.ds(base, TPW)])


def _cgather(y, s1, s2):
    mesh = plsc.VectorSubcoreMesh(core_axis_name="c", subcore_axis_name="s")
    out = jax.ShapeDtypeStruct((NT, DM), jnp.float32)
    return pl.kernel(
        _cgather_body,
        mesh=mesh,
        out_type=(out, out),
        scratch_types=[
            pltpu.VMEM((TPW,), jnp.int32),
            pltpu.VMEM((TPW, DM), jnp.float32),
            pltpu.SemaphoreType.DMA,
        ],
    )(y, s1, s2)


def _combine_body(x_ref, g1_ref, g2_ref, sc1_ref, sc2_ref, o_ref):
    o_ref[...] = (x_ref[...] + sc1_ref[...] * g1_ref[...]
                  + sc2_ref[...] * g2_ref[...])


CB = 256


def _combine(x, g1, g2, sc1, sc2):
    row_spec = pl.BlockSpec((CB, DM), lambda i: (i, 0))
    col_spec = pl.BlockSpec((CB, 1), lambda i: (i, 0))
    return pl.pallas_call(
        _combine_body,
        grid=(NT // CB,),
        in_specs=[row_spec, row_spec, row_spec, col_spec, col_spec],
        out_specs=row_spec,
        out_shape=jax.ShapeDtypeStruct((NT, DM), jnp.float32),
    )(x, g1, g2, sc1, sc2)


def kernel(x, Wg, bg, W1, b1, W2, b2):
    (sc1, sc2, d1, d2, s1, s2, cnt) = _gate(x, Wg, bg)
    buf = _dispatch(x, d1.reshape(NT), d2.reshape(NT))
    y = _ffn(cnt, buf, W1, b1, W2, b2)
    g1, g2 = _cgather(y, s1.reshape(NT), s2.reshape(NT))
    return _combine(x, g1, g2, sc1, sc2)


# speedup vs baseline: 1.7371x; 1.0102x over previous
import functools

import jax
import jax.numpy as jnp
from jax import lax
from jax.experimental import pallas as pl
from jax.experimental.pallas import tpu as pltpu
from jax.experimental.pallas import tpu_sc as plsc

DM = 1024
DI = 1024
NE = 64
KK = 2
NT = 2048
CAP = 256
BUF_ROWS = NE * CAP + CAP

NW = 32
TPW = NT // NW


GB = 512


def _gate_body(x_ref, wg_ref, bg_ref, sc1_ref, sc2_ref,
               d1_ref, d2_ref, s1_ref, s2_ref, cnt_ref, carry_ref):
    g = pl.program_id(0)

    @pl.when(g == 0)
    def _():
        carry_ref[...] = jnp.zeros((1, NE), jnp.float32)

    x = x_ref[...]
    logits = jax.lax.dot_general(
        x, wg_ref[...], (((1,), (0,)), ((), ())),
        preferred_element_type=jnp.float32) + bg_ref[...]
    ii = lax.broadcasted_iota(jnp.int32, (GB, NE), 1)
    v1 = jnp.max(logits, axis=1, keepdims=True)
    e1 = jnp.min(jnp.where(logits == v1, ii, NE + 1), axis=1, keepdims=True)
    l2 = jnp.where(ii == e1, -jnp.inf, logits)
    v2 = jnp.max(l2, axis=1, keepdims=True)
    e2 = jnp.min(jnp.where(l2 == v2, ii, NE + 1), axis=1, keepdims=True)
    t = jnp.exp(v2 - v1)
    s1 = 1.0 / (1.0 + t)
    s2 = 1.0 - s1
    oh1 = (ii == e1).astype(jnp.float32)
    oh2 = (ii == e2).astype(jnp.float32)
    m = oh1 + oh2
    r_i = lax.broadcasted_iota(jnp.int32, (GB, GB), 0)
    c_i = lax.broadcasted_iota(jnp.int32, (GB, GB), 1)
    ltri = (c_i < r_i).astype(jnp.float32)
    carry = carry_ref[...]
    pex = jax.lax.dot_general(
        ltri, m, (((1,), (0,)), ((), ())),
        preferred_element_type=jnp.float32) + carry
    new_carry = carry + jnp.sum(m, axis=0, keepdims=True)
    carry_ref[...] = new_carry
    r1 = jnp.sum(pex * oh1, axis=1, keepdims=True).astype(jnp.int32)
    r2 = jnp.sum(pex * oh2, axis=1, keepdims=True).astype(jnp.int32)
    ok1 = r1 < CAP
    ok2 = r2 < CAP
    row1 = e1 * CAP + r1
    row2 = e2 * CAP + r2
    sc1_ref[...] = jnp.where(ok1, s1, 0.0)
    sc2_ref[...] = jnp.where(ok2, s2, 0.0)
    d1_ref[...] = jnp.squeeze(jnp.where(ok1, row1, NE * CAP), 1)
    d2_ref[...] = jnp.squeeze(jnp.where(ok2, row2, NE * CAP), 1)
    s1_ref[...] = jnp.squeeze(e1 * CAP + jnp.minimum(r1, CAP - 1), 1)
    s2_ref[...] = jnp.squeeze(e2 * CAP + jnp.minimum(r2, CAP - 1), 1)
    cnt_ref[...] = new_carry.astype(jnp.int32)


def _gate(x, wg, bg):
    ng = NT // GB
    flat_i = jax.ShapeDtypeStruct((NT,), jnp.int32)
    col_f = jax.ShapeDtypeStruct((NT, 1), jnp.float32)
    col_spec = pl.BlockSpec((GB, 1), lambda g: (g, 0))
    flat_spec = pl.BlockSpec((GB,), lambda g: (g,))
    return pl.pallas_call(
        _gate_body,
        grid=(ng,),
        in_specs=[
            pl.BlockSpec((GB, DM), lambda g: (g, 0)),
            pl.BlockSpec((DM, NE), lambda g: (0, 0)),
            pl.BlockSpec((1, NE), lambda g: (0, 0)),
        ],
        out_specs=(col_spec, col_spec, flat_spec, flat_spec,
                   flat_spec, flat_spec, pl.BlockSpec((1, NE), lambda g: (0, 0))),
        out_shape=(col_f, col_f, flat_i, flat_i, flat_i, flat_i,
                   jax.ShapeDtypeStruct((1, NE), jnp.int32)),
        scratch_shapes=[pltpu.VMEM((1, NE), jnp.float32)],
    )(x, wg, bg.reshape(1, NE))


def _dispatch_body(x_hbm, d1_hbm, d2_hbm, buf_hbm, idx_v, rows_v, sem):
    wid = lax.axis_index("s") * 2 + lax.axis_index("c")
    base = wid * TPW
    pltpu.sync_copy(x_hbm.at[pl.ds(base, TPW)], rows_v)
    pltpu.sync_copy(d1_hbm.at[pl.ds(base, TPW)], idx_v)
    pltpu.async_copy(rows_v, buf_hbm.at[idx_v], sem).wait()
    pltpu.sync_copy(d2_hbm.at[pl.ds(base, TPW)], idx_v)
    pltpu.async_copy(rows_v, buf_hbm.at[idx_v], sem).wait()


def _dispatch(x, d1, d2):
    mesh = plsc.VectorSubcoreMesh(core_axis_name="c", subcore_axis_name="s")
    return pl.kernel(
        _dispatch_body,
        mesh=mesh,
        out_type=jax.ShapeDtypeStruct((BUF_ROWS, DM), jnp.float32),
        scratch_types=[
            pltpu.VMEM((TPW,), jnp.int32),
            pltpu.VMEM((TPW, DM), jnp.float32),
            pltpu.SemaphoreType.DMA,
        ],
    )(x, d1, d2)


RT = 64
NST = CAP // RT


def _ffn_body(cnt_ref, buf_hbm, w1_ref, b1_ref, w2_ref, b2_ref, y_hbm,
              xb_v, y_v, lsem, ssem):
    e = pl.program_id(0)
    slot = lax.rem(e, 2)

    def ld(ee, s, j):
        return pltpu.make_async_copy(
            buf_hbm.at[pl.ds(ee * CAP + j * RT, RT), :],
            xb_v.at[s, pl.ds(j * RT, RT), :], lsem.at[s, j])

    def st(ee, s, j):
        return pltpu.make_async_copy(
            y_v.at[s, pl.ds(j * RT, RT), :],
            y_hbm.at[pl.ds(ee * CAP + j * RT, RT), :], ssem.at[s, j])

    def issue_loads(ee, s):
        cn = cnt_ref[0, ee]
        for j in range(NST):
            @pl.when(cn > j * RT)
            def _(j=j):
                ld(ee, s, j).start()

    @pl.when(e == 0)
    def _():
        issue_loads(0, 0)

    @pl.when(e + 1 < NE)
    def _():
        issue_loads(e + 1, 1 - slot)

    @pl.when(e >= 2)
    def _():
        cp = cnt_ref[0, e - 2]
        for j in range(NST):
            @pl.when(cp > j * RT)
            def _(j=j):
                st(e - 2, slot, j).wait()

    cnt = cnt_ref[0, e]
    for j in range(NST):
        @pl.when(cnt > j * RT)
        def _(j=j):
            ld(e, slot, j).wait()
            h = jax.lax.dot_general(
                xb_v[slot, pl.ds(j * RT, RT), :], w1_ref[0],
                (((1,), (0,)), ((), ())),
                preferred_element_type=jnp.float32) + b1_ref[pl.ds(e, 1), :]
            h = jnp.maximum(h, 0.0)
            y = jax.lax.dot_general(
                h, w2_ref[0], (((1,), (0,)), ((), ())),
                preferred_element_type=jnp.float32) + b2_ref[pl.ds(e, 1), :]
            y_v[slot, pl.ds(j * RT, RT), :] = y
            st(e, slot, j).start()

    @pl.when(e == NE - 1)
    def _():
        for ee in (NE - 2, NE - 1):
            cp = cnt_ref[0, ee]
            for j in range(NST):
                @pl.when(cp > j * RT)
                def _(ee=ee, j=j):
                    st(ee, ee % 2, j).wait()


def _ffn(cnt, buf, w1, b1, w2, b2):
    return pl.pallas_call(
        _ffn_body,
        grid=(NE,),
        in_specs=[
            pl.BlockSpec(memory_space=pltpu.SMEM),
            pl.BlockSpec(memory_space=pl.ANY),
            pl.BlockSpec((1, DM, DI), lambda e: (e, 0, 0)),
            pl.BlockSpec((NE, DI), lambda e: (0, 0)),
            pl.BlockSpec((1, DI, DM), lambda e: (e, 0, 0)),
            pl.BlockSpec((NE, DM), lambda e: (0, 0)),
        ],
        out_specs=pl.BlockSpec(memory_space=pl.ANY),
        out_shape=jax.ShapeDtypeStruct((NE * CAP, DM), jnp.float32),
        scratch_shapes=[
            pltpu.VMEM((2, CAP, DM), jnp.float32),
            pltpu.VMEM((2, CAP, DM), jnp.float32),
            pltpu.SemaphoreType.DMA((2, NST)),
            pltpu.SemaphoreType.DMA((2, NST)),
        ],
    )(cnt, buf, w1, b1, w2, b2)


def _cgather_body(y_hbm, s1_hbm, s2_hbm, g1_hbm, g2_hbm, idx_v, rows_v, sem):
    wid = lax.axis_index("s") * 2 + lax.axis_index("c")
    base = wid * TPW
    pltpu.sync_copy(s1_hbm.at[pl.ds(base, TPW)], idx_v)
    pltpu.async_copy(y_hbm.at[idx_v], rows_v, sem).wait()
    pltpu.sync_copy(rows_v, g1_hbm.at[pl.ds(base, TPW)])
    pltpu.sync_copy(s2_hbm.at[pl.ds(base, TPW)], idx_v)
    pltpu.async_copy(y_hbm.at[idx_v], rows_v, sem).wait()
    pltpu.sync_copy(rows_v, g2_hbm.at[pl.ds(base, TPW)])


def _cgather(y, s1, s2):
    mesh = plsc.VectorSubcoreMesh(core_axis_name="c", subcore_axis_name="s")
    out = jax.ShapeDtypeStruct((NT, DM), jnp.float32)
    return pl.kernel(
        _cgather_body,
        mesh=mesh,
        out_type=(out, out),
        scratch_types=[
            pltpu.VMEM((TPW,), jnp.int32),
            pltpu.VMEM((TPW, DM), jnp.float32),
            pltpu.SemaphoreType.DMA,
        ],
    )(y, s1, s2)


def _combine_body(x_ref, g1_ref, g2_ref, sc1_ref, sc2_ref, o_ref):
    o_ref[...] = (x_ref[...] + sc1_ref[...] * g1_ref[...]
                  + sc2_ref[...] * g2_ref[...])


CB = 256


def _combine(x, g1, g2, sc1, sc2):
    row_spec = pl.BlockSpec((CB, DM), lambda i: (i, 0))
    col_spec = pl.BlockSpec((CB, 1), lambda i: (i, 0))
    return pl.pallas_call(
        _combine_body,
        grid=(NT // CB,),
        in_specs=[row_spec, row_spec, row_spec, col_spec, col_spec],
        out_specs=row_spec,
        out_shape=jax.ShapeDtypeStruct((NT, DM), jnp.float32),
    )(x, g1, g2, sc1, sc2)


def kernel(x, Wg, bg, W1, b1, W2, b2):
    (sc1, sc2, d1, d2, s1, s2, cnt) = _gate(x, Wg, bg)
    buf = _dispatch(x, d1, d2)
    y = _ffn(cnt, buf, W1, b1, W2, b2)
    g1, g2 = _cgather(y, s1, s2)
    return _combine(x, g1, g2, sc1, sc2)
